# Initial kernel scaffold; baseline (speedup 1.0000x reference)
import jax
import jax.numpy as jnp
from jax.experimental import pallas as pl


def kernel(h, edge_index, edge_attr, batch, W, b):
    raise NotImplementedError("write your pallas kernel here")


# speedup vs baseline: 1.2014x; 1.2014x over previous
import functools

import jax
import jax.numpy as jnp
from jax import lax
from jax.experimental import pallas as pl
from jax.experimental.pallas import tpu as pltpu
from jax.experimental.pallas import tpu_sc as plsc

N = 4096
KK = 2048
D = 256

NC = 2
NS = 16
NW = NC * NS
RPW = KK // NW
EPW = N // NS


def _rank_body(s_ref, rank_ref):
    s = s_ref[...]
    col = s[:, None]
    i_iota = lax.broadcasted_iota(jnp.int32, (N, 1), 0)

    def body(jc, acc):
        chunk = s_ref[pl.ds(jc * 128, 128)]
        row = chunk[None, :]
        j_iota = lax.broadcasted_iota(jnp.int32, (1, 128), 1) + jc * 128
        gt = row > col
        eq = (row == col) & (j_iota < i_iota)
        return acc + jnp.sum((gt | eq).astype(jnp.int32), axis=1)

    rank_ref[...] = lax.fori_loop(0, N // 128, body, jnp.zeros((N,), jnp.int32))


def _compute_ranks(s):
    return pl.pallas_call(
        _rank_body,
        out_shape=jax.ShapeDtypeStruct((N,), jnp.int32),
        in_specs=[pl.BlockSpec(memory_space=pltpu.VMEM)],
    )(s)


def _cast_body(a_ref, o_ref):
    o_ref[...] = a_ref[...].astype(jnp.bfloat16)


def _cast_adj(a):
    return pl.pallas_call(
        _cast_body,
        grid=(8,),
        in_specs=[pl.BlockSpec((N // 8, N), lambda i: (i, 0))],
        out_specs=pl.BlockSpec((N // 8, N), lambda i: (i, 0)),
        out_shape=jax.ShapeDtypeStruct((N, N), jnp.bfloat16),
    )(a)


def _sc_body(s_hbm, rank_hbm, ab_hbm, h_hbm, idx_hbm, val_hbm, b_hbm, hr_hbm,
             s_v, rank_v, tgt_v, src_v, idx_w, val_w, rows_v, hrows_v,
             idx_sh, val_sh, sem):
    cid = lax.axis_index("c")
    sid = lax.axis_index("s")
    wid = sid * NC + cid

    base_e = sid * EPW
    pltpu.sync_copy(rank_hbm.at[pl.ds(base_e, EPW)], rank_v)
    pltpu.sync_copy(s_hbm.at[pl.ds(base_e, EPW)], s_v)
    for i in range(EPW // 16):
        r = rank_v[pl.ds(i * 16, 16)]
        tgt_v[i // 8, pl.ds((i % 8) * 16, 16)] = jnp.minimum(r, KK)
        src_v[pl.ds(i * 16, 16)] = lax.iota(jnp.int32, 16) + (base_e + i * 16)
    for c in range(2):
        pltpu.sync_copy(src_v.at[pl.ds(c * 128, 128)], idx_sh.at[tgt_v.at[c]])
        pltpu.sync_copy(s_v.at[pl.ds(c * 128, 128)], val_sh.at[tgt_v.at[c]])
    plsc.subcore_barrier()

    base = wid * RPW
    pltpu.sync_copy(idx_sh.at[pl.ds(base, RPW)], idx_w)
    pltpu.sync_copy(val_sh.at[pl.ds(base, RPW)], val_w)
    pltpu.sync_copy(idx_w, idx_hbm.at[pl.ds(base, RPW)])
    pltpu.sync_copy(val_w, val_hbm.at[pl.ds(base, RPW)])

    for cc in range(RPW // 16):
        ii = idx_w.at[pl.ds(cc * 16, 16)]
        pltpu.async_copy(ab_hbm.at[ii], rows_v, sem).wait()
        pltpu.sync_copy(rows_v, b_hbm.at[pl.ds(base + cc * 16, 16)])

    pltpu.async_copy(h_hbm.at[idx_w], hrows_v, sem).wait()
    pltpu.sync_copy(hrows_v, hr_hbm.at[pl.ds(base, RPW)])


def _sc_select_gather(s, rank, a_i32, h):
    mesh = plsc.VectorSubcoreMesh(core_axis_name="c", subcore_axis_name="s",
                                  num_cores=NC, num_subcores=NS)
    f = pl.kernel(
        _sc_body,
        out_type=(
            jax.ShapeDtypeStruct((KK,), jnp.int32),
            jax.ShapeDtypeStruct((KK,), jnp.float32),
            jax.ShapeDtypeStruct((KK, N), jnp.int32),
            jax.ShapeDtypeStruct((KK, D), jnp.float32),
        ),
        mesh=mesh,
        scratch_types=[
            pltpu.VMEM((EPW,), jnp.float32),
            pltpu.VMEM((EPW,), jnp.int32),
            pltpu.VMEM((2, EPW // 2), jnp.int32),
            pltpu.VMEM((EPW,), jnp.int32),
            pltpu.VMEM((RPW,), jnp.int32),
            pltpu.VMEM((RPW,), jnp.float32),
            pltpu.VMEM((16, N), jnp.int32),
            pltpu.VMEM((RPW, D), jnp.float32),
            pltpu.VMEM_SHARED((KK + 8,), jnp.int32),
            pltpu.VMEM_SHARED((KK + 8,), jnp.float32),
            pltpu.SemaphoreType.DMA,
        ],
    )
    return f(s, rank, a_i32, h)


def _bigmm_body(b_ref, a_ref, u_ref, acc_ref):
    m = pl.program_id(1)

    @pl.when(m == 0)
    def _():
        acc_ref[...] = jnp.zeros_like(acc_ref)

    acc_ref[...] += jnp.dot(b_ref[...].astype(jnp.bfloat16), a_ref[...],
                            preferred_element_type=jnp.float32)

    @pl.when(m == pl.num_programs(1) - 1)
    def _():
        u_ref[...] = (acc_ref[...] != 0.0).astype(jnp.bfloat16)


def _bigmm(b_rows, a_bf16):
    bm, bk = 512, 512
    return pl.pallas_call(
        _bigmm_body,
        grid=(KK // bm, N // bk),
        in_specs=[
            pl.BlockSpec((bm, bk), lambda p, m: (p, m)),
            pl.BlockSpec((bk, N), lambda p, m: (m, 0)),
        ],
        out_specs=pl.BlockSpec((bm, N), lambda p, m: (p, 0)),
        out_shape=jax.ShapeDtypeStruct((KK, N), jnp.bfloat16),
        scratch_shapes=[pltpu.VMEM((bm, N), jnp.float32)],
    )(b_rows, a_bf16)


def _colsel_body(u_ref, idx_ref, ung_ref, gout_ref):
    j = pl.program_id(0)
    nj = pl.num_programs(0)
    bj = N // nj
    j_iota = lax.broadcasted_iota(jnp.int32, (bj, 1), 0) + j * bj
    oh = (j_iota == idx_ref[...][None, :]).astype(jnp.bfloat16)

    @pl.when(j == 0)
    def _():
        ung_ref[...] = jnp.zeros_like(ung_ref)

    ung_ref[...] += jnp.dot(u_ref[...], oh, preferred_element_type=jnp.float32)

    @pl.when(j == nj - 1)
    def _():
        ung = ung_ref[...]
        deg = jnp.sum(ung, axis=1)
        gout_ref[...] = ung / deg[None, :]


def _colselect(u, idx):
    bj = 512
    return pl.pallas_call(
        _colsel_body,
        grid=(N // bj,),
        in_specs=[
            pl.BlockSpec((KK, bj), lambda j: (0, j)),
            pl.BlockSpec(memory_space=pltpu.VMEM),
        ],
        out_specs=(
            pl.BlockSpec((KK, KK), lambda j: (0, 0)),
            pl.BlockSpec((KK, KK), lambda j: (0, 0)),
        ),
        out_shape=(
            jax.ShapeDtypeStruct((KK, KK), jnp.float32),
            jax.ShapeDtypeStruct((KK, KK), jnp.float32),
        ),
    )(u, idx)


def _newh_body(hr_ref, val_ref, o_ref):
    o_ref[...] = hr_ref[...] * val_ref[...][:, None]


def _new_h(h_rows, values):
    return pl.pallas_call(
        _newh_body,
        out_shape=jax.ShapeDtypeStruct((KK, D), jnp.float32),
    )(h_rows, values)


def kernel(h, edge_index, edge_attr, batch, W, b):
    s = jax.nn.sigmoid((h @ W.T + b).squeeze(-1))
    rank = _compute_ranks(s)
    a_bf16 = _cast_adj(edge_index)
    idx, values, b_rows, h_rows = _sc_select_gather(s, rank, edge_index, h)
    u = _bigmm(b_rows, a_bf16)
    un_g, g_out = _colselect(u, idx)
    new_h = _new_h(h_rows, values)
    new_batch = jnp.zeros((KK,), dtype=jnp.int32)
    return (g_out, new_h, idx, un_g, un_g, new_batch)


# speedup vs baseline: 1.2625x; 1.0508x over previous
import functools

import jax
import jax.numpy as jnp
from jax import lax
from jax.experimental import pallas as pl
from jax.experimental.pallas import tpu as pltpu
from jax.experimental.pallas import tpu_sc as plsc

N = 4096
KK = 2048
D = 256

NC = 2
NS = 16
NW = NC * NS
RPW = KK // NW
EPW = N // NS


def _rank_body(s_ref, idx_ref, val_ref):
    s = s_ref[...]
    col = s[:, None]
    i_iota = lax.broadcasted_iota(jnp.int32, (N, 1), 0)

    def body(jc, acc):
        chunk = s_ref[pl.ds(jc * 128, 128)]
        row = chunk[None, :]
        j_iota = lax.broadcasted_iota(jnp.int32, (1, 128), 1) + jc * 128
        gt = row > col
        eq = (row == col) & (j_iota < i_iota)
        return acc + jnp.sum((gt | eq).astype(jnp.int32), axis=1)

    rank = lax.fori_loop(0, N // 128, body, jnp.zeros((N,), jnp.int32))
    rank_col = rank[:, None]

    def sel(pc, _):
        p_row = lax.broadcasted_iota(jnp.int32, (1, 128), 1) + pc * 128
        onehot = rank_col == p_row
        idx_ref[pl.ds(pc * 128, 128)] = jnp.sum(
            jnp.where(onehot, i_iota, 0), axis=0)
        val_ref[pl.ds(pc * 128, 128)] = jnp.sum(
            jnp.where(onehot, col, 0.0), axis=0)
        return 0

    lax.fori_loop(0, KK // 128, sel, 0)


def _rank_select(s):
    return pl.pallas_call(
        _rank_body,
        out_shape=(
            jax.ShapeDtypeStruct((KK,), jnp.int32),
            jax.ShapeDtypeStruct((KK,), jnp.float32),
        ),
        in_specs=[pl.BlockSpec(memory_space=pltpu.VMEM)],
    )(s)


def _sc_body(idx_hbm, ab_hbm, h_hbm, b_hbm, hr_hbm,
             idx_w, rows_v, hrows_v, sem):
    cid = lax.axis_index("c")
    sid = lax.axis_index("s")
    wid = sid * NC + cid

    base = wid * RPW
    pltpu.sync_copy(idx_hbm.at[pl.ds(base, RPW)], idx_w)

    for cc in range(RPW // 16):
        ii = idx_w.at[pl.ds(cc * 16, 16)]
        pltpu.async_copy(ab_hbm.at[ii], rows_v, sem).wait()
        pltpu.sync_copy(rows_v, b_hbm.at[pl.ds(base + cc * 16, 16)])

    pltpu.async_copy(h_hbm.at[idx_w], hrows_v, sem).wait()
    pltpu.sync_copy(hrows_v, hr_hbm.at[pl.ds(base, RPW)])


def _sc_gather(idx, a_i32, h):
    mesh = plsc.VectorSubcoreMesh(core_axis_name="c", subcore_axis_name="s",
                                  num_cores=NC, num_subcores=NS)
    f = pl.kernel(
        _sc_body,
        out_type=(
            jax.ShapeDtypeStruct((KK, N), jnp.int32),
            jax.ShapeDtypeStruct((KK, D), jnp.float32),
        ),
        mesh=mesh,
        scratch_types=[
            pltpu.VMEM((RPW,), jnp.int32),
            pltpu.VMEM((16, N), jnp.int32),
            pltpu.VMEM((RPW, D), jnp.float32),
            pltpu.SemaphoreType.DMA,
        ],
    )
    return f(idx, a_i32, h)


def _castb_body(b_ref, o_ref):
    o_ref[...] = b_ref[...].astype(jnp.bfloat16)


def _cast_b(b_rows):
    return pl.pallas_call(
        _castb_body,
        grid=(4,),
        in_specs=[pl.BlockSpec((KK // 4, N), lambda i: (i, 0))],
        out_specs=pl.BlockSpec((KK // 4, N), lambda i: (i, 0)),
        out_shape=jax.ShapeDtypeStruct((KK, N), jnp.bfloat16),
    )(b_rows)


def _bigmm_body(bb_ref, a_ref, u_ref):
    t = jnp.dot(bb_ref[...], a_ref[...].astype(jnp.bfloat16),
                preferred_element_type=jnp.float32)
    u_ref[...] = (t != 0.0).astype(jnp.bfloat16)


def _bigmm(bb, a_i32):
    bq = 512
    return pl.pallas_call(
        _bigmm_body,
        grid=(N // bq,),
        in_specs=[
            pl.BlockSpec((KK, N), lambda q: (0, 0)),
            pl.BlockSpec((N, bq), lambda q: (0, q)),
        ],
        out_specs=pl.BlockSpec((KK, bq), lambda q: (0, q)),
        out_shape=jax.ShapeDtypeStruct((KK, N), jnp.bfloat16),
    )(bb, a_i32)


def _colsel_body(u_ref, idx_ref, ung_ref, gout_ref):
    j = pl.program_id(0)
    nj = pl.num_programs(0)
    bj = N // nj
    j_iota = lax.broadcasted_iota(jnp.int32, (bj, 1), 0) + j * bj
    oh = (j_iota == idx_ref[...][None, :]).astype(jnp.bfloat16)

    @pl.when(j == 0)
    def _():
        ung_ref[...] = jnp.zeros_like(ung_ref)

    ung_ref[...] += jnp.dot(u_ref[...], oh, preferred_element_type=jnp.float32)

    @pl.when(j == nj - 1)
    def _():
        ung = ung_ref[...]
        deg = jnp.sum(ung, axis=1)
        gout_ref[...] = ung / deg[None, :]


def _colselect(u, idx):
    bj = 512
    return pl.pallas_call(
        _colsel_body,
        grid=(N // bj,),
        in_specs=[
            pl.BlockSpec((KK, bj), lambda j: (0, j)),
            pl.BlockSpec(memory_space=pltpu.VMEM),
        ],
        out_specs=(
            pl.BlockSpec((KK, KK), lambda j: (0, 0)),
            pl.BlockSpec((KK, KK), lambda j: (0, 0)),
        ),
        out_shape=(
            jax.ShapeDtypeStruct((KK, KK), jnp.float32),
            jax.ShapeDtypeStruct((KK, KK), jnp.float32),
        ),
    )(u, idx)


def _newh_body(hr_ref, val_ref, o_ref):
    o_ref[...] = hr_ref[...] * val_ref[...][:, None]


def _new_h(h_rows, values):
    return pl.pallas_call(
        _newh_body,
        out_shape=jax.ShapeDtypeStruct((KK, D), jnp.float32),
    )(h_rows, values)


def kernel(h, edge_index, edge_attr, batch, W, b):
    s = jax.nn.sigmoid((h @ W.T + b).squeeze(-1))
    idx, values = _rank_select(s)
    b_rows, h_rows = _sc_gather(idx, edge_index, h)
    u = _bigmm(_cast_b(b_rows), edge_index)
    un_g, g_out = _colselect(u, idx)
    new_h = _new_h(h_rows, values)
    new_batch = jnp.zeros((KK,), dtype=jnp.int32)
    return (g_out, new_h, idx, un_g, un_g, new_batch)
